# baseline (speedup 1.0000x reference)
import functools

import jax
import jax.numpy as jnp
from jax import lax
from jax.experimental import pallas as pl
from jax.experimental.pallas import tpu as pltpu
from jax.experimental.pallas import tpu_sc as plsc

N_CLASSES = 1000000
M_DIM = 32
BATCH = 16384

NUM_CORES = 2
NUM_SUBCORES = 16
LANES = 16
NUM_WORKERS = NUM_CORES * NUM_SUBCORES
B_PER_W = BATCH // NUM_WORKERS
PACK = 128 // M_DIM
N_SUPER = N_CLASSES // PACK
CHUNK = 128
N_CHUNKS = B_PER_W // CHUNK


def _rsqrt_newton(s):
  i = lax.bitcast_convert_type(s, jnp.int32)
  i = jnp.int32(0x5F3759DF) - lax.shift_right_logical(i, 1)
  y = lax.bitcast_convert_type(i, jnp.float32)
  half = s * 0.5
  for _ in range(3):
    y = y * (1.5 - half * y * y)
  return y


@jax.jit
def _embed_norm(idx, table4):
  mesh = plsc.VectorSubcoreMesh(core_axis_name="c", subcore_axis_name="s")

  @functools.partial(
      pl.kernel,
      out_type=jax.ShapeDtypeStruct((BATCH, M_DIM), jnp.float32),
      mesh=mesh,
      scratch_types=[
          pltpu.VMEM((B_PER_W,), jnp.int32),
          pltpu.VMEM((B_PER_W,), jnp.int32),
          pltpu.VMEM((CHUNK, 4 * M_DIM), jnp.float32),
          pltpu.VMEM((B_PER_W, M_DIM), jnp.float32),
          pltpu.SemaphoreType.DMA,
      ],
      compiler_params=pltpu.CompilerParams(
          needs_layout_passes=False, use_tc_tiling_on_sc=True
      ),
  )
  def k(idx_hbm, table_hbm, out_hbm, idx_v, idxq_v, rows_v, out_v, sem):
    wid = lax.axis_index("s") * NUM_CORES + lax.axis_index("c")
    base = wid * B_PER_W
    pltpu.sync_copy(idx_hbm.at[pl.ds(base, B_PER_W)], idx_v)

    @pl.loop(0, B_PER_W // LANES)
    def _(j):
      v = idx_v[pl.ds(j * LANES, LANES)]
      idxq_v[pl.ds(j * LANES, LANES)] = lax.shift_right_logical(v, 2)

    @pl.loop(0, N_CHUNKS)
    def _(c):
      cbase = c * CHUNK
      pltpu.async_copy(
          table_hbm.at[idxq_v.at[pl.ds(cbase, CHUNK)]], rows_v, sem
      ).wait()

      @pl.loop(0, CHUNK // LANES)
      def _(g):
        iv = idx_v[pl.ds(cbase + g * LANES, LANES)]
        offv = (iv & 3) * M_DIM
        for l in range(LANES):
          r = g * LANES + l
          o = offv[l]
          v0 = rows_v[r, pl.ds(o, LANES)]
          v1 = rows_v[r, pl.ds(o + LANES, LANES)]
          ss = v0 * v0 + v1 * v1
          tot = jnp.broadcast_to(jnp.sum(ss), (LANES,))
          y = _rsqrt_newton(tot)
          out_v[cbase + r, pl.ds(0, LANES)] = v0 * y
          out_v[cbase + r, pl.ds(LANES, LANES)] = v1 * y

    pltpu.sync_copy(out_v, out_hbm.at[pl.ds(base, B_PER_W)])

  return k(idx, table4)


def kernel(x, table):
  table4 = table.reshape(N_SUPER, PACK * M_DIM)
  return _embed_norm(x.astype(jnp.int32), table4)


# speedup vs baseline: 1.6614x; 1.6614x over previous
import functools

import jax
import jax.numpy as jnp
from jax import lax
from jax.experimental import pallas as pl
from jax.experimental.pallas import tpu as pltpu
from jax.experimental.pallas import tpu_sc as plsc

N_CLASSES = 1000000
M_DIM = 32
BATCH = 16384

NUM_CORES = 2
NUM_SUBCORES = 16
LANES = 16
NUM_WORKERS = NUM_CORES * NUM_SUBCORES
B_PER_W = BATCH // NUM_WORKERS
CHUNK = 64
N_CHUNKS = B_PER_W // CHUNK


def _rsqrt_newton(s):
  i = lax.bitcast_convert_type(s, jnp.int32)
  i = jnp.int32(0x5F3759DF) - lax.shift_right_logical(i, 1)
  y = lax.bitcast_convert_type(i, jnp.float32)
  half = s * 0.5
  for _ in range(3):
    y = y * (1.5 - half * y * y)
  return y


@jax.jit
def _embed_norm(idx, table):
  mesh = plsc.VectorSubcoreMesh(core_axis_name="c", subcore_axis_name="s")

  @functools.partial(
      pl.kernel,
      out_type=jax.ShapeDtypeStruct((BATCH, M_DIM), jnp.float32),
      mesh=mesh,
      scratch_types=[
          pltpu.VMEM((B_PER_W,), jnp.int32),
          pltpu.VMEM((CHUNK, M_DIM), jnp.float32),
          pltpu.VMEM((CHUNK, M_DIM), jnp.float32),
          pltpu.SemaphoreType.DMA,
      ],
      compiler_params=pltpu.CompilerParams(
          needs_layout_passes=False, use_tc_tiling_on_sc=True
      ),
  )
  def k(idx_hbm, table_hbm, out_hbm, idx_v, rows_v, out_v, sem):
    wid = lax.axis_index("s") * NUM_CORES + lax.axis_index("c")
    base = wid * B_PER_W
    pltpu.sync_copy(idx_hbm.at[pl.ds(base, B_PER_W)], idx_v)

    @pl.loop(0, N_CHUNKS)
    def _(c):
      cbase = c * CHUNK
      copies = []
      for g in range(CHUNK // LANES):
        iv = idx_v[pl.ds(cbase + g * LANES, LANES)]
        for l in range(LANES):
          r = g * LANES + l
          copies.append(
              pltpu.async_copy(table_hbm.at[iv[l]], rows_v.at[r], sem)
          )
      for cp in copies:
        cp.wait()

      for g in range(CHUNK // LANES):
        for l in range(LANES):
          r = g * LANES + l
          v0 = rows_v[r, pl.ds(0, LANES)]
          v1 = rows_v[r, pl.ds(LANES, LANES)]
          ss = v0 * v0 + v1 * v1
          tot = jnp.broadcast_to(jnp.sum(ss), (LANES,))
          y = _rsqrt_newton(tot)
          out_v[r, pl.ds(0, LANES)] = v0 * y
          out_v[r, pl.ds(LANES, LANES)] = v1 * y

      pltpu.sync_copy(out_v, out_hbm.at[pl.ds(base + cbase, CHUNK)])

  return k(idx, table)


def kernel(x, table):
  return _embed_norm(x.astype(jnp.int32), table)


# speedup vs baseline: 3.3237x; 2.0006x over previous
import functools

import jax
import jax.numpy as jnp
from jax import lax
from jax.experimental import pallas as pl
from jax.experimental.pallas import tpu as pltpu
from jax.experimental.pallas import tpu_sc as plsc

N_CLASSES = 1000000
M_DIM = 32
BATCH = 16384

NUM_CORES = 2
NUM_SUBCORES = 16
LANES = 16
NUM_WORKERS = NUM_CORES * NUM_SUBCORES
B_PER_W = BATCH // NUM_WORKERS
CHUNK = 16
N_CHUNKS = B_PER_W // CHUNK


def _rsqrt_newton(s):
  i = lax.bitcast_convert_type(s, jnp.int32)
  i = jnp.int32(0x5F3759DF) - lax.shift_right_logical(i, 1)
  y = lax.bitcast_convert_type(i, jnp.float32)
  half = s * 0.5
  for _ in range(3):
    y = y * (1.5 - half * y * y)
  return y


@jax.jit
def _embed_norm_t(idx, table_t):
  mesh = plsc.VectorSubcoreMesh(core_axis_name="c", subcore_axis_name="s")

  @functools.partial(
      pl.kernel,
      out_type=jax.ShapeDtypeStruct((M_DIM, BATCH), jnp.float32),
      mesh=mesh,
      scratch_types=[
          pltpu.VMEM((B_PER_W,), jnp.int32),
          pltpu.VMEM((CHUNK, M_DIM, 128), jnp.float32),
          pltpu.VMEM((M_DIM, B_PER_W), jnp.float32),
          pltpu.SemaphoreType.DMA,
      ],
      compiler_params=pltpu.CompilerParams(
          needs_layout_passes=False, use_tc_tiling_on_sc=True
      ),
  )
  def k(idx_hbm, table_hbm, out_hbm, idx_v, blk_v, out_v, sem):
    wid = lax.axis_index("s") * NUM_CORES + lax.axis_index("c")
    base = wid * B_PER_W
    pltpu.sync_copy(idx_hbm.at[pl.ds(base, B_PER_W)], idx_v)
    lane_ids = lax.iota(jnp.int32, LANES)

    @pl.loop(0, N_CHUNKS)
    def _(c):
      iv = idx_v[pl.ds(c * CHUNK, CHUNK)]
      copies = []
      for l in range(CHUNK):
        off = pl.multiple_of(
            lax.shift_right_logical(iv[l], 7) * 128, 128
        )
        copies.append(
            pltpu.async_copy(
                table_hbm.at[:, pl.ds(off, 128)], blk_v.at[l], sem
            )
        )
      for cp in copies:
        cp.wait()

      sub = iv & 127
      for l in range(CHUNK):
        j = jnp.broadcast_to(sub[l], (LANES,))
        rsel = jnp.broadcast_to(jnp.int32(l), (LANES,))
        v0 = plsc.load_gather(blk_v, [rsel, lane_ids, j])
        v1 = plsc.load_gather(blk_v, [rsel, lane_ids + LANES, j])
        ss = v0 * v0 + v1 * v1
        tot = jnp.broadcast_to(jnp.sum(ss), (LANES,))
        y = _rsqrt_newton(tot)
        col = jnp.broadcast_to(c * CHUNK + l, (LANES,))
        plsc.store_scatter(out_v, [lane_ids, col], v0 * y)
        plsc.store_scatter(out_v, [lane_ids + LANES, col], v1 * y)

    pltpu.sync_copy(out_v, out_hbm.at[:, pl.ds(base, B_PER_W)])

  return k(idx, table_t)


def kernel(x, table):
  out_t = _embed_norm_t(x.astype(jnp.int32), table.T)
  return out_t.T


# speedup vs baseline: 3.8505x; 1.1585x over previous
import functools

import jax
import jax.numpy as jnp
from jax import lax
from jax.experimental import pallas as pl
from jax.experimental.pallas import tpu as pltpu
from jax.experimental.pallas import tpu_sc as plsc

N_CLASSES = 1000000
M_DIM = 32
BATCH = 16384

NUM_CORES = 2
NUM_SUBCORES = 16
LANES = 16
NUM_WORKERS = NUM_CORES * NUM_SUBCORES
B_PER_W = BATCH // NUM_WORKERS
CHUNK = 8
N_PAIRS = B_PER_W // LANES


def _rsqrt_newton(s):
  i = lax.bitcast_convert_type(s, jnp.int32)
  i = jnp.int32(0x5F3759DF) - lax.shift_right_logical(i, 1)
  y = lax.bitcast_convert_type(i, jnp.float32)
  half = s * 0.5
  for _ in range(3):
    y = y * (1.5 - half * y * y)
  return y


@jax.jit
def _embed_norm_t(idx, table_t):
  mesh = plsc.VectorSubcoreMesh(core_axis_name="c", subcore_axis_name="s")

  @functools.partial(
      pl.kernel,
      out_type=jax.ShapeDtypeStruct((M_DIM, BATCH), jnp.float32),
      mesh=mesh,
      scratch_types=[
          pltpu.VMEM((B_PER_W,), jnp.int32),
          pltpu.VMEM((CHUNK, M_DIM, 128), jnp.float32),
          pltpu.VMEM((CHUNK, M_DIM, 128), jnp.float32),
          pltpu.VMEM((M_DIM, B_PER_W), jnp.float32),
          pltpu.SemaphoreType.DMA,
          pltpu.SemaphoreType.DMA,
      ],
      compiler_params=pltpu.CompilerParams(
          needs_layout_passes=False, use_tc_tiling_on_sc=True
      ),
  )
  def k(idx_hbm, table_hbm, out_hbm, idx_v, blk_a, blk_b, out_v, sem_a, sem_b):
    wid = lax.axis_index("s") * NUM_CORES + lax.axis_index("c")
    base = wid * B_PER_W
    pltpu.sync_copy(idx_hbm.at[pl.ds(base, B_PER_W)], idx_v)
    lane_ids = lax.iota(jnp.int32, LANES)

    def issue(buf, sem, p, h):
      iv = idx_v[pl.ds(p * LANES, LANES)]
      for l in range(CHUNK):
        off = pl.multiple_of(
            lax.shift_right_logical(iv[h * CHUNK + l], 7) * 128, 128
        )
        pltpu.async_copy(table_hbm.at[:, pl.ds(off, 128)], buf.at[l], sem)

    def drain(buf, sem):
      for l in range(CHUNK):
        pltpu.make_async_copy(
            table_hbm.at[:, pl.ds(0, 128)], buf.at[l], sem
        ).wait()

    def compute(buf, p, h):
      iv = idx_v[pl.ds(p * LANES, LANES)]
      sub = iv & 127
      for l in range(CHUNK):
        j = jnp.broadcast_to(sub[h * CHUNK + l], (LANES,))
        rsel = jnp.broadcast_to(jnp.int32(l), (LANES,))
        v0 = plsc.load_gather(buf, [rsel, lane_ids, j])
        v1 = plsc.load_gather(buf, [rsel, lane_ids + LANES, j])
        ss = v0 * v0 + v1 * v1
        tot = jnp.broadcast_to(jnp.sum(ss), (LANES,))
        y = _rsqrt_newton(tot)
        col = jnp.broadcast_to(p * LANES + h * CHUNK + l, (LANES,))
        plsc.store_scatter(out_v, [lane_ids, col], v0 * y)
        plsc.store_scatter(out_v, [lane_ids + LANES, col], v1 * y)

    issue(blk_a, sem_a, 0, 0)

    @pl.loop(0, N_PAIRS)
    def _(i):
      issue(blk_b, sem_b, i, 1)
      drain(blk_a, sem_a)
      compute(blk_a, i, 0)

      @pl.when(i < N_PAIRS - 1)
      def _():
        issue(blk_a, sem_a, i + 1, 0)

      drain(blk_b, sem_b)
      compute(blk_b, i, 1)

    pltpu.sync_copy(out_v, out_hbm.at[:, pl.ds(base, B_PER_W)])

  return k(idx, table_t)


def kernel(x, table):
  out_t = _embed_norm_t(x.astype(jnp.int32), table.T)
  return out_t.T
